# baseline (speedup 1.0000x reference)
import functools

import jax
import jax.numpy as jnp
from jax import lax
from jax.experimental import pallas as pl
from jax.experimental.pallas import tpu as pltpu
from jax.experimental.pallas import tpu_sc as plsc

NBINS = 256
L = 16
NC = 2
NS = 16
NW = NC * NS
UNROLL = 8
NBUF = 2
CHUNK = 8192


def _make_sc_bucketize(n: int, chunk: int, nbuf: int):
    assert n % (NW * chunk) == 0
    per_w = n // NW
    nchunk = per_w // chunk
    assert nchunk % nbuf == 0
    rounds = nchunk // nbuf

    mesh = plsc.VectorSubcoreMesh(core_axis_name="c", subcore_axis_name="s")

    @functools.partial(
        pl.kernel,
        mesh=mesh,
        out_type=jax.ShapeDtypeStruct((n,), jnp.int32),
        scratch_types=(
            [pltpu.VMEM_SHARED((NS, nbuf, chunk), jnp.float32),
             pltpu.VMEM_SHARED((NS, nbuf, chunk), jnp.int32)]
            + [pltpu.VMEM((chunk,), jnp.float32) for _ in range(nbuf)]
            + [pltpu.VMEM((chunk,), jnp.int32) for _ in range(nbuf)]
            + [pltpu.VMEM((2 * L,), jnp.float32)]
            + [pltpu.SemaphoreType.DMA for _ in range(4 * nbuf)]
        ),
    )
    def sc_bucketize(x_hbm, consts_hbm, y_hbm, sp_in, sp_out, *bufs):
        inb = bufs[:nbuf]
        outb = bufs[nbuf:2 * nbuf]
        cv = bufs[2 * nbuf]
        sems = bufs[2 * nbuf + 1:]
        asem = sems[:nbuf]
        bsem = sems[nbuf:2 * nbuf]
        csem = sems[2 * nbuf:3 * nbuf]
        dsem = sems[3 * nbuf:4 * nbuf]

        sid = lax.axis_index("s")
        wid = sid * NC + lax.axis_index("c")
        base = wid * per_w

        pltpu.sync_copy(consts_hbm, cv)
        scale = cv[pl.ds(0, L)]
        beta = cv[pl.ds(L, L)]
        zero = jnp.zeros((L,), jnp.int32)
        top = jnp.full((L,), NBINS, jnp.int32)

        def compute(src, dst):
            @plsc.parallel_loop(0, chunk, step=L, unroll=UNROLL)
            def _(o):
                v = src[pl.ds(o, L)]
                t = v * scale + beta
                k = t.astype(jnp.int32)
                k = jnp.minimum(k, top)
                k = jnp.maximum(k, zero)
                dst[pl.ds(o, L)] = k

        def start_a(c, b):
            off = pl.multiple_of(base + c * chunk, 8)
            pltpu.async_copy(
                x_hbm.at[pl.ds(off, chunk)], sp_in.at[sid, b], asem[b])

        def wait_a(b):
            pltpu.make_async_copy(
                x_hbm.at[pl.ds(0, chunk)], sp_in.at[sid, b], asem[b]).wait()

        def copy_b(b):
            pltpu.async_copy(sp_in.at[sid, b], inb[b], bsem[b]).wait()

        def copy_c(b):
            pltpu.async_copy(outb[b], sp_out.at[sid, b], csem[b]).wait()

        def start_d(b, c):
            off = pl.multiple_of(base + c * chunk, 8)
            pltpu.async_copy(
                sp_out.at[sid, b], y_hbm.at[pl.ds(off, chunk)], dsem[b])

        def wait_d(b):
            pltpu.make_async_copy(
                sp_out.at[sid, b], y_hbm.at[pl.ds(0, chunk)], dsem[b]).wait()

        for b in range(nbuf):
            start_a(b, b)

        def round_body(q, carry):
            for b in range(nbuf):
                c = q * nbuf + b
                wait_a(b)
                copy_b(b)
                compute(inb[b], outb[b])

                @pl.when(q > 0)
                def _():
                    wait_d(b)

                copy_c(b)
                start_d(b, c)

                @pl.when(q < rounds - 1)
                def _():
                    start_a(c + nbuf, b)
            return carry

        lax.fori_loop(0, rounds, round_body, 0)
        for b in range(nbuf):
            wait_d(b)

    return sc_bucketize


def kernel(x, fMin, fMax):
    n = x.size
    xf = x.reshape(n)
    scale = jnp.float32(NBINS - 1) / (fMax - fMin)
    beta = jnp.float32(1.0) - fMin * scale
    consts = jnp.concatenate([
        jnp.full((L,), scale, jnp.float32),
        jnp.full((L,), beta, jnp.float32),
    ])
    y = _make_sc_bucketize(n, CHUNK, NBUF)(xf, consts)
    return y.reshape(x.shape).astype(jnp.int64)


# speedup vs baseline: 2.0157x; 2.0157x over previous
import functools

import jax
import jax.numpy as jnp
from jax import lax
from jax.experimental import pallas as pl
from jax.experimental.pallas import tpu as pltpu
from jax.experimental.pallas import tpu_sc as plsc

NBINS = 256
L = 16
NC = 2
NS = 16
NW = NC * NS
UNROLL = 8
NBUF = 2
ROWS = 128


def _make_sc_bucketize(n_rows: int, cols: int):
    assert n_rows % (NW * ROWS) == 0
    rows_per_w = n_rows // NW
    nchunk = rows_per_w // ROWS
    assert nchunk % NBUF == 0
    rounds = nchunk // NBUF
    groups = cols // L

    mesh = plsc.VectorSubcoreMesh(core_axis_name="c", subcore_axis_name="s")

    @functools.partial(
        pl.kernel,
        mesh=mesh,
        out_type=jax.ShapeDtypeStruct((n_rows, cols), jnp.int32),
        compiler_params=pltpu.CompilerParams(use_tc_tiling_on_sc=True),
        scratch_types=(
            [pltpu.VMEM((ROWS, cols), jnp.float32) for _ in range(NBUF)]
            + [pltpu.VMEM((ROWS, cols), jnp.int32) for _ in range(NBUF)]
            + [pltpu.VMEM((2 * L,), jnp.float32)]
            + [pltpu.SemaphoreType.DMA for _ in range(2 * NBUF)]
        ),
    )
    def sc_bucketize(x_hbm, consts_hbm, y_hbm, *bufs):
        inb = bufs[:NBUF]
        outb = bufs[NBUF:2 * NBUF]
        cv = bufs[2 * NBUF]
        isem = bufs[2 * NBUF + 1:2 * NBUF + 1 + NBUF]
        osem = bufs[2 * NBUF + 1 + NBUF:]

        wid = lax.axis_index("s") * NC + lax.axis_index("c")
        base = wid * rows_per_w

        pltpu.sync_copy(consts_hbm, cv)
        scale = cv[pl.ds(0, L)]
        beta = cv[pl.ds(L, L)]
        zero = jnp.zeros((L,), jnp.int32)
        top = jnp.full((L,), NBINS, jnp.int32)

        def compute(src, dst):
            @plsc.parallel_loop(0, ROWS, step=1, unroll=UNROLL)
            def _(r):
                for g in range(groups):
                    v = src[r, pl.ds(g * L, L)]
                    t = v * scale + beta
                    k = t.astype(jnp.int32)
                    k = jnp.minimum(k, top)
                    k = jnp.maximum(k, zero)
                    dst[r, pl.ds(g * L, L)] = k

        def start_in(c, b):
            off = pl.multiple_of(base + c * ROWS, 8)
            pltpu.async_copy(
                x_hbm.at[pl.ds(off, ROWS), :], inb[b], isem[b])

        def wait_in(b):
            pltpu.make_async_copy(
                x_hbm.at[pl.ds(0, ROWS), :], inb[b], isem[b]).wait()

        def start_out(b, c):
            off = pl.multiple_of(base + c * ROWS, 8)
            pltpu.async_copy(
                outb[b], y_hbm.at[pl.ds(off, ROWS), :], osem[b])

        def wait_out(b):
            pltpu.make_async_copy(
                outb[b], y_hbm.at[pl.ds(0, ROWS), :], osem[b]).wait()

        for b in range(NBUF):
            start_in(b, b)

        def round_body(q, carry):
            for b in range(NBUF):
                c = q * NBUF + b
                wait_in(b)

                @pl.when(q > 0)
                def _():
                    wait_out(b)

                compute(inb[b], outb[b])
                start_out(b, c)

                @pl.when(q < rounds - 1)
                def _():
                    start_in(c + NBUF, b)
            return carry

        lax.fori_loop(0, rounds, round_body, 0)
        for b in range(NBUF):
            wait_out(b)

    return sc_bucketize


def kernel(x, fMin, fMax):
    rows = x.shape[0] * x.shape[1]
    cols = x.shape[2]
    x2 = x.reshape(rows, cols)
    scale = jnp.float32(NBINS - 1) / (fMax - fMin)
    beta = jnp.float32(1.0) - fMin * scale
    consts = jnp.concatenate([
        jnp.full((L,), scale, jnp.float32),
        jnp.full((L,), beta, jnp.float32),
    ])
    y = _make_sc_bucketize(rows, cols)(x2, consts)
    return y.reshape(x.shape).astype(jnp.int64)
